# baseline (speedup 1.0000x reference)
import functools

import jax
import jax.numpy as jnp
from jax import lax
from jax.experimental import pallas as pl
from jax.experimental.pallas import tpu as pltpu
from jax.experimental.pallas import tpu_sc as plsc

BATCH = 16384
FEATURE_DIM = 64
NUM_CLASSES = 100000
PAIR = 50176
PROWS = PAIR // 8
HIST = 100352
NC = 2
NS = 16
NW = NC * NS
PER_TILE = BATCH // NW
NCHUNK = PER_TILE // 128
HROWS = 8

_mesh = plsc.VectorSubcoreMesh(core_axis_name="c", subcore_axis_name="s")


def _tc_repack(t0_ref, t1_ref, o_ref):
    o_ref[:, :FEATURE_DIM] = t0_ref[...].T
    o_ref[:, FEATURE_DIM:] = t1_ref[...].T


def _make_table(centers_t):
    return pl.pallas_call(
        _tc_repack,
        grid=(4,),
        in_specs=[
            pl.BlockSpec((FEATURE_DIM, PAIR // 4), lambda i: (0, i)),
            pl.BlockSpec((FEATURE_DIM, PAIR // 4), lambda i: (0, i + 4)),
        ],
        out_specs=pl.BlockSpec((PAIR // 4, 128), lambda i: (i, 0)),
        out_shape=jax.ShapeDtypeStruct((PAIR, 128), jnp.float32),
    )(centers_t, centers_t)


@functools.partial(
    pl.kernel,
    mesh=_mesh,
    out_type=jax.ShapeDtypeStruct((NW, NCHUNK, 128), jnp.float32),
    scratch_types=(
        pltpu.VMEM((NCHUNK, 128), jnp.int32),
        pltpu.VMEM((HROWS, 128), jnp.int32),
        pltpu.VMEM((NCHUNK, 128), jnp.float32),
        pltpu.VMEM((128,), jnp.float32),
        pltpu.VMEM((128,), jnp.float32),
        pltpu.VMEM_SHARED((HIST,), jnp.float32),
    ),
    compiler_params=pltpu.CompilerParams(use_tc_tiling_on_sc=True),
)
def _sc_count(label2d_hbm, cnt_hbm,
              lab_v, labh_v, cnt_v, zeros_v, ones_v, hist_sh):
    cid = lax.axis_index("c")
    sid = lax.axis_index("s")
    wid = sid * NC + cid

    pltpu.sync_copy(label2d_hbm.at[pl.ds(wid * NCHUNK, NCHUNK)], lab_v)
    pltpu.sync_copy(label2d_hbm.at[pl.ds(sid * HROWS, HROWS)], labh_v)
    for k in range(128 // 16):
        zeros_v[pl.ds(k * 16, 16)] = jnp.zeros((16,), jnp.float32)
        ones_v[pl.ds(k * 16, 16)] = jnp.ones((16,), jnp.float32)

    for j in range(HROWS):
        pltpu.sync_copy(zeros_v, hist_sh.at[labh_v.at[j]])
    plsc.subcore_barrier()
    for j in range(HROWS):
        pltpu.sync_copy(ones_v, hist_sh.at[labh_v.at[j]], add=True)
    plsc.subcore_barrier()

    for j in range(NCHUNK):
        pltpu.sync_copy(hist_sh.at[lab_v.at[j]], cnt_v.at[j])
    for j in range(NCHUNK):
        for g in range(128 // 16):
            lab = lab_v[j, pl.ds(g * 16, 16)]
            hi = (((lab - PAIR) >> 31) & 1).astype(jnp.float32)
            cnt_v[j, pl.ds(g * 16, 16)] *= 2.0 * hi - 1.0
    pltpu.sync_copy(cnt_v, cnt_hbm.at[wid])


@functools.partial(
    pl.kernel,
    mesh=_mesh,
    out_type=jax.ShapeDtypeStruct((BATCH, 128), jnp.float32),
    scratch_types=(
        pltpu.VMEM((NCHUNK, 128), jnp.int32),
        pltpu.VMEM((NCHUNK, 128), jnp.int32),
        pltpu.VMEM((PER_TILE, 128), jnp.float32),
        pltpu.SemaphoreType.DMA,
    ),
    compiler_params=pltpu.CompilerParams(use_tc_tiling_on_sc=True),
)
def _sc_gather(label2d_hbm, table_hbm, gath_hbm, lab_v, idxp_v, pair_buf, sem):
    cid = lax.axis_index("c")
    sid = lax.axis_index("s")
    wid = sid * NC + cid
    base = wid * PER_TILE

    pltpu.sync_copy(label2d_hbm.at[pl.ds(wid * NCHUNK, NCHUNK)], lab_v)
    for j in range(NCHUNK):
        for g in range(128 // 16):
            lab = lab_v[j, pl.ds(g * 16, 16)]
            notm = ((lab - PAIR) >> 31) ^ (-1)
            idxp_v[j, pl.ds(g * 16, 16)] = lab - (notm & PAIR)

    copies = [
        pltpu.async_copy(table_hbm.at[idxp_v.at[c]],
                         pair_buf.at[pl.ds(c * 128, 128)], sem)
        for c in range(NCHUNK)
    ]
    for c, cp in enumerate(copies):
        cp.wait()
        pltpu.sync_copy(pair_buf.at[pl.ds(c * 128, 128)],
                        gath_hbm.at[pl.ds(base + c * 128, 128)])


def _tc_loss(f_ref, g_ref, c_ref, o_ref):
    i = pl.program_id(0)

    @pl.when(i == 0)
    def _():
        o_ref[...] = jnp.zeros((1, 1), jnp.float32)

    f = f_ref[...]
    gp = g_ref[...]
    cols = c_ref[...]
    lane = jax.lax.broadcasted_iota(jnp.int32, (_BLK, _GRID), 1)
    c = jnp.sum(jnp.where(lane == i, cols, 0.0), axis=1, keepdims=True)
    d0 = f - gp[:, :FEATURE_DIM]
    d1 = f - gp[:, FEATURE_DIM:]
    s0 = jnp.sum(d0 * d0, axis=1, keepdims=True)
    s1 = jnp.sum(d1 * d1, axis=1, keepdims=True)
    s = jnp.where(c < 0.0, s1, s0)
    o_ref[...] += (jnp.sum(jnp.sqrt(s / jnp.abs(c))) * (1.0 / BATCH)).reshape(1, 1)


_GRID = 4
_BLK = BATCH // _GRID


def kernel(feature, label, centers):
    label2d = label.reshape(BATCH // 128, 128)
    cnt3 = _sc_count(label2d)
    table = _make_table(centers.T)
    gath = _sc_gather(label2d, table)
    cntc = cnt3.reshape(_GRID, _BLK).T
    loss = pl.pallas_call(
        _tc_loss,
        grid=(_GRID,),
        in_specs=[
            pl.BlockSpec((_BLK, FEATURE_DIM), lambda i: (i, 0)),
            pl.BlockSpec((_BLK, 128), lambda i: (i, 0)),
            pl.BlockSpec((_BLK, _GRID), lambda i: (0, 0)),
        ],
        out_specs=pl.BlockSpec((1, 1), lambda i: (0, 0)),
        out_shape=jax.ShapeDtypeStruct((1, 1), jnp.float32),
    )(feature, gath, cntc)
    return loss[0, 0]


# speedup vs baseline: 1.0113x; 1.0113x over previous
import functools

import jax
import jax.numpy as jnp
from jax import lax
from jax.experimental import pallas as pl
from jax.experimental.pallas import tpu as pltpu
from jax.experimental.pallas import tpu_sc as plsc

BATCH = 16384
FEATURE_DIM = 64
NUM_CLASSES = 100000
PAIR = 50176
PROWS = PAIR // 8
HIST = 100352
NC = 2
NS = 16
NW = NC * NS
PER_TILE = BATCH // NW
NCHUNK = PER_TILE // 128
HROWS = 8

_mesh = plsc.VectorSubcoreMesh(core_axis_name="c", subcore_axis_name="s")


def _tc_repack(t0_ref, t1_ref, o_ref):
    o_ref[:, :FEATURE_DIM] = t0_ref[...].T
    o_ref[:, FEATURE_DIM:] = t1_ref[...].T


def _make_table(centers_t):
    return pl.pallas_call(
        _tc_repack,
        grid=(4,),
        in_specs=[
            pl.BlockSpec((FEATURE_DIM, PAIR // 4), lambda i: (0, i)),
            pl.BlockSpec((FEATURE_DIM, PAIR // 4), lambda i: (0, i + 4)),
        ],
        out_specs=pl.BlockSpec((PAIR // 4, 128), lambda i: (i, 0)),
        out_shape=jax.ShapeDtypeStruct((PAIR, 128), jnp.float32),
    )(centers_t, centers_t)


@functools.partial(
    pl.kernel,
    mesh=_mesh,
    out_type=jax.ShapeDtypeStruct((NW, NCHUNK, 128), jnp.float32),
    scratch_types=(
        pltpu.VMEM((NCHUNK, 128), jnp.int32),
        pltpu.VMEM((HROWS, 128), jnp.int32),
        pltpu.VMEM((NCHUNK, 128), jnp.float32),
        pltpu.VMEM((128,), jnp.float32),
        pltpu.VMEM((128,), jnp.float32),
        pltpu.VMEM_SHARED((HIST,), jnp.float32),
    ),
    compiler_params=pltpu.CompilerParams(use_tc_tiling_on_sc=True),
)
def _sc_count(label2d_hbm, cnt_hbm,
              lab_v, labh_v, cnt_v, zeros_v, ones_v, hist_sh):
    cid = lax.axis_index("c")
    sid = lax.axis_index("s")
    wid = sid * NC + cid

    pltpu.sync_copy(label2d_hbm.at[pl.ds(wid * NCHUNK, NCHUNK)], lab_v)
    pltpu.sync_copy(label2d_hbm.at[pl.ds(sid * HROWS, HROWS)], labh_v)
    for k in range(128 // 16):
        zeros_v[pl.ds(k * 16, 16)] = jnp.zeros((16,), jnp.float32)
        ones_v[pl.ds(k * 16, 16)] = jnp.ones((16,), jnp.float32)

    for j in range(HROWS):
        pltpu.sync_copy(zeros_v, hist_sh.at[labh_v.at[j]])
    plsc.subcore_barrier()
    for j in range(HROWS):
        pltpu.sync_copy(ones_v, hist_sh.at[labh_v.at[j]], add=True)
    plsc.subcore_barrier()

    for j in range(NCHUNK):
        pltpu.sync_copy(hist_sh.at[lab_v.at[j]], cnt_v.at[j])
    for j in range(NCHUNK):
        for g in range(128 // 16):
            lab = lab_v[j, pl.ds(g * 16, 16)]
            hi = (((lab - PAIR) >> 31) & 1).astype(jnp.float32)
            cnt_v[j, pl.ds(g * 16, 16)] *= 2.0 * hi - 1.0
    pltpu.sync_copy(cnt_v, cnt_hbm.at[wid])


@functools.partial(
    pl.kernel,
    mesh=_mesh,
    out_type=jax.ShapeDtypeStruct((BATCH, 128), jnp.float32),
    scratch_types=(
        pltpu.VMEM((NCHUNK, 128), jnp.int32),
        pltpu.VMEM((NCHUNK, 128), jnp.int32),
        pltpu.VMEM((PER_TILE, 128), jnp.float32),
        pltpu.SemaphoreType.DMA,
    ),
    compiler_params=pltpu.CompilerParams(use_tc_tiling_on_sc=True),
)
def _sc_gather(label2d_hbm, table_hbm, gath_hbm, lab_v, idxp_v, pair_buf, sem):
    cid = lax.axis_index("c")
    sid = lax.axis_index("s")
    wid = sid * NC + cid
    base = wid * PER_TILE

    pltpu.sync_copy(label2d_hbm.at[pl.ds(wid * NCHUNK, NCHUNK)], lab_v)
    for j in range(NCHUNK):
        for g in range(128 // 16):
            lab = lab_v[j, pl.ds(g * 16, 16)]
            notm = ((lab - PAIR) >> 31) ^ (-1)
            idxp_v[j, pl.ds(g * 16, 16)] = lab - (notm & PAIR)

    copies = [
        pltpu.async_copy(table_hbm.at[idxp_v.at[c]],
                         pair_buf.at[pl.ds(c * 128, 128)], sem)
        for c in range(NCHUNK)
    ]
    for c, cp in enumerate(copies):
        cp.wait()
        pltpu.sync_copy(pair_buf.at[pl.ds(c * 128, 128)],
                        gath_hbm.at[pl.ds(base + c * 128, 128)])


def _tc_loss(f_ref, g_ref, c_ref, o_ref):
    i = pl.program_id(0)

    @pl.when(i == 0)
    def _():
        o_ref[...] = jnp.zeros((1, 1), jnp.float32)

    f = f_ref[...]
    gp = g_ref[...]
    cols = c_ref[...]
    lane = jax.lax.broadcasted_iota(jnp.int32, (_BLK, _GRID), 1)
    c = jnp.sum(jnp.where(lane == i, cols, 0.0), axis=1, keepdims=True)
    d0 = f - gp[:, :FEATURE_DIM]
    d1 = f - gp[:, FEATURE_DIM:]
    s0 = jnp.sum(d0 * d0, axis=1, keepdims=True)
    s1 = jnp.sum(d1 * d1, axis=1, keepdims=True)
    s = jnp.where(c < 0.0, s1, s0)
    o_ref[...] += (jnp.sum(jnp.sqrt(s / jnp.abs(c))) * (1.0 / BATCH)).reshape(1, 1)


_GRID = 8
_BLK = BATCH // _GRID


def kernel(feature, label, centers):
    label2d = label.reshape(BATCH // 128, 128)
    cnt3 = _sc_count(label2d)
    table = _make_table(centers.T)
    gath = _sc_gather(label2d, table)
    cntc = cnt3.reshape(_GRID, _BLK).T
    loss = pl.pallas_call(
        _tc_loss,
        grid=(_GRID,),
        in_specs=[
            pl.BlockSpec((_BLK, FEATURE_DIM), lambda i: (i, 0)),
            pl.BlockSpec((_BLK, 128), lambda i: (i, 0)),
            pl.BlockSpec((_BLK, _GRID), lambda i: (0, 0)),
        ],
        out_specs=pl.BlockSpec((1, 1), lambda i: (0, 0)),
        out_shape=jax.ShapeDtypeStruct((1, 1), jnp.float32),
    )(feature, gath, cntc)
    return loss[0, 0]
